# Initial kernel scaffold; baseline (speedup 1.0000x reference)
import jax
import jax.numpy as jnp
from jax.experimental import pallas as pl


def kernel(x, edge_index, edge_type, node_type, emb, edge_type_emb, node_type_bias, W_ih, W_hh, b_ih, b_hh):
    raise NotImplementedError("write your pallas kernel here")


# speedup vs baseline: 4.2401x; 4.2401x over previous
import functools

import jax
import jax.numpy as jnp
from jax import lax
from jax.experimental import pallas as pl
from jax.experimental.pallas import tpu as pltpu
from jax.experimental.pallas import tpu_sc as plsc

N = 10000
E = 320000
D = 128
T = 16
NC = 2
NS = 16
NW = NC * NS
NPAD = 10240
RPT = NPAD // NW
RROWS = NPAD // NS
EC = 128
EPT = 10112
EPAD = NW * EPT
NCHUNK = EPT // EC
GCHUNK = 80

def _mesh():
    return plsc.VectorSubcoreMesh(
        core_axis_name="c", subcore_axis_name="s", num_cores=NC, num_subcores=NS)


def _worker_id():
    return lax.axis_index("s") * NC + lax.axis_index("c")


@functools.cache
def _row_gather():
    @functools.partial(
        pl.kernel,
        out_type=jax.ShapeDtypeStruct((NPAD, D), jnp.float32),
        mesh=_mesh(),
        scratch_types=[
            pltpu.VMEM((GCHUNK,), jnp.int32),
            pltpu.VMEM((GCHUNK, D), jnp.float32),
            pltpu.SemaphoreType.DMA,
        ],
    )
    def k(table_hbm, idx_hbm, out_hbm, idx_v, rows_v, sem):
        base = _worker_id() * RPT
        for j in range(RPT // GCHUNK):
            off = base + j * GCHUNK
            pltpu.sync_copy(idx_hbm.at[pl.ds(off, GCHUNK)], idx_v)
            pltpu.async_copy(table_hbm.at[idx_v], rows_v, sem).wait()
            pltpu.sync_copy(rows_v, out_hbm.at[pl.ds(off, GCHUNK)])

    return k


@functools.cache
def _sc_degree():
    @functools.partial(
        pl.kernel,
        out_type=jax.ShapeDtypeStruct((NC, NPAD, D), jnp.float32),
        mesh=_mesh(),
        scratch_types=[
            pltpu.VMEM((EC,), jnp.int32),
            pltpu.VMEM((EC, D), jnp.float32),
            pltpu.VMEM_SHARED((NPAD, D), jnp.float32),
            pltpu.SemaphoreType.DMA,
        ],
    )
    def k(dst_hbm, ones_hbm, zeros_hbm, out_hbm, didx_v, ones_v, deg_sh, sem):
        c = lax.axis_index("c")
        s = lax.axis_index("s")
        w = s * NC + c
        pltpu.sync_copy(zeros_hbm.at[pl.ds(s * RROWS, RROWS)],
                        deg_sh.at[pl.ds(s * RROWS, RROWS)])
        pltpu.sync_copy(ones_hbm, ones_v)
        plsc.subcore_barrier()
        ebase = w * EPT

        def chunk(i, carry):
            pltpu.sync_copy(dst_hbm.at[pl.ds(ebase + i * EC, EC)], didx_v)
            pltpu.sync_copy(ones_v, deg_sh.at[didx_v], add=True)
            return carry

        lax.fori_loop(0, NCHUNK, chunk, 0)
        plsc.subcore_barrier()
        pltpu.sync_copy(deg_sh.at[pl.ds(s * RROWS, RROWS)],
                        out_hbm.at[c, pl.ds(s * RROWS, RROWS)])

    return k


@functools.cache
def _sc_aggregate():
    @functools.partial(
        pl.kernel,
        out_type=jax.ShapeDtypeStruct((NC, NPAD, D), jnp.float32),
        mesh=_mesh(),
        scratch_types=[
            pltpu.VMEM((2, EC), jnp.int32),
            pltpu.VMEM((EC,), jnp.int32),
            pltpu.VMEM((EC,), jnp.int32),
            pltpu.VMEM((EC, D), jnp.float32),
            pltpu.VMEM((EC, D), jnp.float32),
            pltpu.VMEM_SHARED((NPAD, D), jnp.float32),
            pltpu.SemaphoreType.DMA,
            pltpu.SemaphoreType.DMA,
            pltpu.SemaphoreType.DMA,
            pltpu.SemaphoreType.DMA,
            pltpu.SemaphoreType.DMA,
        ],
    )
    def k(h16_hbm, cidx_hbm, dst_hbm, zeros_hbm, out_hbm,
          cidx2_v, didxa_v, didxb_v, rowsa_v, rowsb_v, aggr_sh,
          semia, semib, semga, semgb, semsa):
        c = lax.axis_index("c")
        s = lax.axis_index("s")
        w = s * NC + c
        pltpu.sync_copy(zeros_hbm.at[pl.ds(s * RROWS, RROWS)],
                        aggr_sh.at[pl.ds(s * RROWS, RROWS)])
        plsc.subcore_barrier()
        ebase = w * EPT

        def pair(j, carry):
            offa = ebase + (2 * j) * EC
            offb = offa + EC
            ia0 = pltpu.async_copy(cidx_hbm.at[pl.ds(offa, EC)],
                                   cidx2_v.at[0], semia)
            ia1 = pltpu.async_copy(dst_hbm.at[pl.ds(offa, EC)], didxa_v, semia)
            ib0 = pltpu.async_copy(cidx_hbm.at[pl.ds(offb, EC)],
                                   cidx2_v.at[1], semib)
            ib1 = pltpu.async_copy(dst_hbm.at[pl.ds(offb, EC)], didxb_v, semib)
            ia0.wait()
            ia1.wait()
            ga = pltpu.async_copy(h16_hbm.at[cidx2_v.at[0]], rowsa_v, semga)
            ib0.wait()
            ib1.wait()
            gb = pltpu.async_copy(h16_hbm.at[cidx2_v.at[1]], rowsb_v, semgb)
            ga.wait()
            sa = pltpu.make_async_copy(rowsa_v, aggr_sh.at[didxa_v], semsa)
            sa.start(add=True)
            gb.wait()
            pltpu.sync_copy(rowsb_v, aggr_sh.at[didxb_v], add=True)
            sa.wait()
            return carry

        lax.fori_loop(0, NCHUNK // 2, pair, 0)
        if NCHUNK % 2:
            off = ebase + (NCHUNK - 1) * EC
            pltpu.sync_copy(cidx_hbm.at[pl.ds(off, EC)], cidx2_v.at[0])
            pltpu.sync_copy(dst_hbm.at[pl.ds(off, EC)], didxa_v)
            pltpu.async_copy(h16_hbm.at[cidx2_v.at[0]], rowsa_v, semga).wait()
            pltpu.sync_copy(rowsa_v, aggr_sh.at[didxa_v], add=True)
        plsc.subcore_barrier()
        pltpu.sync_copy(aggr_sh.at[pl.ds(s * RROWS, RROWS)],
                        out_hbm.at[c, pl.ds(s * RROWS, RROWS)])

    return k


_NB = 25
_BR = N // _NB


def _build_body(h_ref, gate_ref, out_ref):
    out_ref[0] = h_ref[...] * gate_ref[0]


_tc_build = pl.pallas_call(
    _build_body,
    grid=(_NB, T),
    in_specs=[
        pl.BlockSpec((_BR, D), lambda i, t: (i, 0)),
        pl.BlockSpec((1, 1, D), lambda i, t: (t, 0, 0)),
    ],
    out_specs=pl.BlockSpec((1, _BR, D), lambda i, t: (t, i, 0)),
    out_shape=jax.ShapeDtypeStruct((T, N, D), jnp.float32),
)


def _gru_core(a_ref, dg_ref, nt_ref, h_ref, wih_ref, whh_ref, bih_ref,
              bhh_ref):
    a = a_ref[0] + a_ref[1]
    dg = dg_ref[0, :, 0:1] + dg_ref[1, :, 0:1]
    dg = jnp.maximum(dg, 1.0)
    inp = a / dg + nt_ref[...]
    h = h_ref[...]
    gi = jnp.dot(inp, wih_ref[...], preferred_element_type=jnp.float32)
    gi = gi + bih_ref[...]
    gh = jnp.dot(h, whh_ref[...], preferred_element_type=jnp.float32)
    gh = gh + bhh_ref[...]
    r = jax.nn.sigmoid(gi[:, 0:D] + gh[:, 0:D])
    z = jax.nn.sigmoid(gi[:, D:2 * D] + gh[:, D:2 * D])
    n = jnp.tanh(gi[:, 2 * D:] + r * gh[:, 2 * D:])
    return (1.0 - z) * n + z * h


def _gru_body(a_ref, dg_ref, nt_ref, h_ref, wih_ref, whh_ref, bih_ref,
              bhh_ref, out_ref):
    out_ref[...] = _gru_core(a_ref, dg_ref, nt_ref, h_ref, wih_ref, whh_ref,
                             bih_ref, bhh_ref)


def _gru_fused_body(a_ref, dg_ref, nt_ref, h_ref, wih_ref, whh_ref, bih_ref,
                    bhh_ref, gate_ref, out_ref, out16_ref):
    hn = _gru_core(a_ref, dg_ref, nt_ref, h_ref, wih_ref, whh_ref,
                   bih_ref, bhh_ref)
    out_ref[...] = hn
    for t in range(T):
        out16_ref[t] = hn * gate_ref[t]


_GRU_SPECS = [
    pl.BlockSpec((NC, _BR, D), lambda i: (0, i, 0)),
    pl.BlockSpec((NC, _BR, D), lambda i: (0, i, 0)),
    pl.BlockSpec((_BR, D), lambda i: (i, 0)),
    pl.BlockSpec((_BR, D), lambda i: (i, 0)),
    pl.BlockSpec((D, 3 * D), lambda i: (0, 0)),
    pl.BlockSpec((D, 3 * D), lambda i: (0, 0)),
    pl.BlockSpec((1, 3 * D), lambda i: (0, 0)),
    pl.BlockSpec((1, 3 * D), lambda i: (0, 0)),
]

_tc_gru = pl.pallas_call(
    _gru_body,
    grid=(_NB,),
    in_specs=_GRU_SPECS,
    out_specs=pl.BlockSpec((_BR, D), lambda i: (i, 0)),
    out_shape=jax.ShapeDtypeStruct((N, D), jnp.float32),
)

_tc_gru_fused = pl.pallas_call(
    _gru_fused_body,
    grid=(_NB,),
    in_specs=_GRU_SPECS + [pl.BlockSpec((T, 1, D), lambda i: (0, 0, 0))],
    out_specs=[
        pl.BlockSpec((_BR, D), lambda i: (i, 0)),
        pl.BlockSpec((T, _BR, D), lambda i: (0, i, 0)),
    ],
    out_shape=[
        jax.ShapeDtypeStruct((N, D), jnp.float32),
        jax.ShapeDtypeStruct((T, N, D), jnp.float32),
    ],
)


def kernel(x, edge_index, edge_type, node_type, emb, edge_type_emb,
           node_type_bias, W_ih, W_hh, b_ih, b_hh):
    src = edge_index[0].astype(jnp.int32)
    dst = edge_index[1].astype(jnp.int32)
    et = edge_type.astype(jnp.int32)
    cidx = et * N + src
    cidx_p = jnp.pad(cidx, (0, EPAD - E))
    dst_p = jnp.pad(dst, (0, EPAD - E), constant_values=N)
    x_p = jnp.pad(x.astype(jnp.int32), (0, NPAD - N))
    nt_p = jnp.pad(node_type.astype(jnp.int32), (0, NPAD - N))
    zeros128 = jnp.zeros((NPAD, D), jnp.float32)
    ones128 = jnp.ones((EC, D), jnp.float32)
    wihT = W_ih.T
    whhT = W_hh.T
    bih2 = b_ih.reshape(1, 3 * D)
    bhh2 = b_hh.reshape(1, 3 * D)

    gather = _row_gather()
    h = gather(emb, x_p)[:N]
    nt_rows = gather(node_type_bias, nt_p)[:N]
    degp = _sc_degree()(dst_p, ones128, zeros128)
    gate3 = edge_type_emb.reshape(T, 1, D)
    aggregate = _sc_aggregate()
    h16 = _tc_build(h, gate3)
    for it in range(5):
        aggp = aggregate(h16.reshape(T * N, D), cidx_p, dst_p, zeros128)
        if it < 4:
            h, h16 = _tc_gru_fused(aggp, degp, nt_rows, h, wihT, whhT,
                                   bih2, bhh2, gate3)
        else:
            h = _tc_gru(aggp, degp, nt_rows, h, wihT, whhT, bih2, bhh2)
    return h
